# Initial kernel scaffold; baseline (speedup 1.0000x reference)
import jax
import jax.numpy as jnp
from jax.experimental import pallas as pl


def kernel(inputs, row_indices, col_indices):
    raise NotImplementedError("write your pallas kernel here")


# speedup vs baseline: 237.9571x; 237.9571x over previous
import functools

import jax
import jax.numpy as jnp
from jax import lax
from jax.experimental import pallas as pl
from jax.experimental.pallas import tpu as pltpu
from jax.experimental.pallas import tpu_sc as plsc

B = 1024
N = 20000
ROWS, COLS = 82, 67
NCELL = ROWS * COLS
CP = 5504
LANES = 16
NW = 32
B_PER_W = B // NW
STAGE = 2000
N_STAGES = N // STAGE
CHUNKS_PER_STAGE = STAGE // LANES
GATHER_CHUNKS = CP // LANES


def _regrid_sc(inputs, row_indices, col_indices):
    mesh = plsc.VectorSubcoreMesh(core_axis_name="c", subcore_axis_name="s")

    @functools.partial(
        pl.kernel,
        mesh=mesh,
        out_type=jax.ShapeDtypeStruct((B, CP), jnp.float32),
        compiler_params=pltpu.CompilerParams(needs_layout_passes=False),
        scratch_types=[
            pltpu.VMEM((LANES * CP,), jnp.int32),
            pltpu.VMEM((CP,), jnp.int32),
            pltpu.VMEM((STAGE,), jnp.int32),
            pltpu.VMEM((STAGE,), jnp.int32),
            pltpu.VMEM((N,), jnp.float32),
            pltpu.VMEM((CP,), jnp.float32),
        ],
    )
    def k(in_hbm, row_hbm, col_hbm, out_hbm, w16, winner, rbuf, cbuf, inbuf, outbuf):
        lane = lax.iota(jnp.int32, LANES)
        wid = lax.axis_index("c") * 16 + lax.axis_index("s")

        neg1 = jnp.full((LANES,), -1, jnp.int32)

        def init_body(i, carry):
            w16[pl.ds(i * LANES, LANES)] = neg1
            return carry

        lax.fori_loop(0, LANES * CP // LANES, init_body, 0)

        for s in range(N_STAGES):
            pltpu.sync_copy(row_hbm.at[pl.ds(s * STAGE, STAGE)], rbuf)
            pltpu.sync_copy(col_hbm.at[pl.ds(s * STAGE, STAGE)], cbuf)

            def scat_body(j, carry, s=s):
                r = rbuf[pl.ds(j * LANES, LANES)]
                c = cbuf[pl.ds(j * LANES, LANES)]
                cell = r * COLS + c
                n_vec = (s * STAGE + j * LANES) + lane
                addr = lane * CP + cell
                plsc.store_scatter(w16, [addr], n_vec)
                return carry

            lax.fori_loop(0, CHUNKS_PER_STAGE, scat_body, 0)

        def red_body(j, carry):
            w = w16[pl.ds(j * LANES, LANES)]
            for l in range(1, LANES):
                w = jnp.maximum(w, w16[pl.ds(l * CP + j * LANES, LANES)])
            winner[pl.ds(j * LANES, LANES)] = w
            return carry

        lax.fori_loop(0, GATHER_CHUNKS, red_body, 0)

        def row_body(bl, carry):
            b = wid * B_PER_W + bl
            pltpu.sync_copy(in_hbm.at[b], inbuf)

            def gat_body(j, carry2):
                idx = winner[pl.ds(j * LANES, LANES)]
                safe = jnp.maximum(idx, 0)
                v = plsc.load_gather(inbuf, [safe])
                v = jnp.where(idx >= 0, v, jnp.zeros((LANES,), jnp.float32))
                outbuf[pl.ds(j * LANES, LANES)] = v
                return carry2

            lax.fori_loop(0, GATHER_CHUNKS, gat_body, 0)
            pltpu.sync_copy(outbuf, out_hbm.at[b])
            return carry

        lax.fori_loop(0, B_PER_W, row_body, 0)

    return k(inputs, row_indices, col_indices)


def kernel(inputs, row_indices, col_indices):
    out = _regrid_sc(inputs, row_indices, col_indices)
    return out[:, :NCELL].reshape(B, ROWS, COLS, 1)


# speedup vs baseline: 308.7059x; 1.2973x over previous
import functools

import jax
import jax.numpy as jnp
from jax import lax
from jax.experimental import pallas as pl
from jax.experimental.pallas import tpu as pltpu
from jax.experimental.pallas import tpu_sc as plsc

B = 1024
N = 20000
ROWS, COLS = 82, 67
NCELL = ROWS * COLS
CP = 5504
LANES = 16
NW = 32
B_PER_W = B // NW
STAGE = 2000
N_STAGES = N // STAGE
CHUNKS_PER_STAGE = STAGE // LANES
GATHER_CHUNKS = CP // LANES


def _regrid_sc(inputs, row_indices, col_indices):
    mesh = plsc.VectorSubcoreMesh(core_axis_name="c", subcore_axis_name="s")

    @functools.partial(
        pl.kernel,
        mesh=mesh,
        out_type=jax.ShapeDtypeStruct((B, CP), jnp.float32),
        compiler_params=pltpu.CompilerParams(needs_layout_passes=False),
        scratch_types=[
            pltpu.VMEM((CP,), jnp.int32),
            pltpu.VMEM((CP,), jnp.float32),
            pltpu.VMEM((STAGE,), jnp.int32),
            pltpu.VMEM((STAGE,), jnp.int32),
            pltpu.VMEM((N,), jnp.float32),
            pltpu.VMEM((N,), jnp.float32),
            pltpu.VMEM((CP,), jnp.float32),
            pltpu.VMEM((CP,), jnp.float32),
            pltpu.SemaphoreType.DMA,
            pltpu.SemaphoreType.DMA,
            pltpu.SemaphoreType.DMA,
            pltpu.SemaphoreType.DMA,
        ],
    )
    def k(in_hbm, row_hbm, col_hbm, out_hbm,
          winner, maskf, rbuf, cbuf, inA, inB, outA, outB,
          isem0, isem1, osem0, osem1):
        lane = lax.iota(jnp.int32, LANES)
        wid = lax.axis_index("c") * 16 + lax.axis_index("s")
        base = wid * B_PER_W
        ins = (inA, inB)
        outs = (outA, outB)
        isems = (isem0, isem1)
        osems = (osem0, osem1)

        for s in range(2):
            pltpu.async_copy(in_hbm.at[base + s], ins[s], isems[s])

        neg1 = jnp.full((LANES,), -1, jnp.int32)

        def init_body(i, carry):
            winner[pl.ds(i * LANES, LANES)] = neg1
            return carry

        lax.fori_loop(0, GATHER_CHUNKS, init_body, 0)

        for s in range(N_STAGES):
            pltpu.sync_copy(row_hbm.at[pl.ds(s * STAGE, STAGE)], rbuf)
            pltpu.sync_copy(col_hbm.at[pl.ds(s * STAGE, STAGE)], cbuf)

            def scat_body(j, carry, s=s):
                r = rbuf[pl.ds(j * LANES, LANES)]
                c = cbuf[pl.ds(j * LANES, LANES)]
                cell = r * COLS + c
                n_vec = (s * STAGE + j * LANES) + lane
                plsc.store_scatter(winner, [cell], n_vec)
                return carry

            lax.fori_loop(0, CHUNKS_PER_STAGE, scat_body, 0)

        def safe_body(j, carry):
            w = winner[pl.ds(j * LANES, LANES)]
            filled = w >= 0
            winner[pl.ds(j * LANES, LANES)] = jnp.maximum(
                w, jnp.zeros((LANES,), jnp.int32))
            maskf[pl.ds(j * LANES, LANES)] = jnp.where(
                filled, jnp.full((LANES,), 1.0, jnp.float32),
                jnp.zeros((LANES,), jnp.float32))
            return carry

        lax.fori_loop(0, GATHER_CHUNKS, safe_body, 0)

        @pl.loop(0, B_PER_W, step=2)
        def pair(bl):
            for s in range(2):
                rl = bl + s
                b = base + rl
                @pl.when(rl >= 2)
                def _():
                    pltpu.make_async_copy(
                        outs[s], out_hbm.at[b - 2], osems[s]).wait()
                pltpu.make_async_copy(in_hbm.at[b], ins[s], isems[s]).wait()

                def gat_body(j, carry, s=s):
                    sl = pl.ds(j * LANES, LANES)
                    idx = winner[sl]
                    outs[s][sl] = plsc.load_gather(ins[s], [idx]) * maskf[sl]
                    return carry

                lax.fori_loop(0, GATHER_CHUNKS, gat_body, 0)

                pltpu.async_copy(outs[s], out_hbm.at[b], osems[s])

                @pl.when(rl + 2 < B_PER_W)
                def _():
                    pltpu.async_copy(in_hbm.at[b + 2], ins[s], isems[s])

        for s in range(2):
            pltpu.make_async_copy(
                outs[s], out_hbm.at[base + B_PER_W - 2 + s], osems[s]).wait()

    return k(inputs, row_indices, col_indices)


def kernel(inputs, row_indices, col_indices):
    out = _regrid_sc(inputs, row_indices, col_indices)
    return out[:, :NCELL].reshape(B, ROWS, COLS, 1)
